# baseline (speedup 1.0000x reference)
import functools

import jax
import jax.numpy as jnp
from jax import lax
from jax.experimental import pallas as pl
from jax.experimental.pallas import tpu as pltpu
from jax.experimental.pallas import tpu_sc as plsc

T = 100
TPAD = 128
NV = TPAD // 32
KTOP = 2
B, D, C, P = 32, 2, 3, 2048
NEG = -2.0
NEG_DUP = 0xC000C000

USC = 5
NU = B * D * C
NU_SC = 32 * USC
NU_TC = NU - NU_SC
CS = P * 2


def _rne_bf16_bits(f):
    u = plsc.bitcast(f, jnp.uint32)
    return (u + jnp.uint32(0x7FFF) + ((u >> jnp.uint32(16)) & jnp.uint32(1))) >> jnp.uint32(16)


def _pl_sc_call(pd_flat):
    mesh = plsc.VectorSubcoreMesh(core_axis_name="c", subcore_axis_name="s")

    slab = C * CS
    uout = KTOP * (TPAD // 2)

    @functools.partial(
        pl.kernel,
        mesh=mesh,
        compiler_params=pltpu.CompilerParams(needs_layout_passes=False),
        out_type=jax.ShapeDtypeStruct((NU_SC * uout,), jnp.uint32),
        scratch_types=[
            pltpu.VMEM((slab,), jnp.float32),
            pltpu.VMEM((USC * uout,), jnp.uint32),
        ],
    )
    def sc_kernel(pd_hbm, out_hbm, in_v, out_v):
        wid = lax.axis_index("s") * 2 + lax.axis_index("c")
        lane2 = lax.iota(jnp.int32, 16).astype(jnp.float32) * 2.0
        tvecs = []
        for j in range(NV):
            te = (lane2 + float(32 * j)) * (1.0 / (T - 1))
            to = (lane2 + float(32 * j + 1)) * (1.0 / (T - 1))
            tu = _rne_bf16_bits(te) | (_rne_bf16_bits(to) << jnp.uint32(16))
            tvecs.append(plsc.bitcast(tu, jnp.bfloat16))

        def dup16(w):
            r = _rne_bf16_bits(w)
            return r | (r << jnp.uint32(16))

        def bc(bits):
            return plsc.bitcast(jnp.full((16,), bits, jnp.uint32),
                                jnp.bfloat16)

        def update(m1, m2, bv, dv):
            nm1, nm2 = [], []
            for j in range(NV):
                v = jnp.minimum(tvecs[j] - bv, dv - tvecs[j])
                nm2.append(jnp.maximum(m2[j], jnp.minimum(m1[j], v)))
                nm1.append(jnp.maximum(m1[j], v))
            return tuple(nm1), tuple(nm2)

        neg = bc(NEG_DUP)
        zero = bc(0)

        for u in range(USC):
            g = wid * USC + u
            dim1 = ((g // 3) % 2) == 1
            pltpu.sync_copy(pd_hbm.at[g // 6, (g // 3) % 2], in_v)
            base = pl.multiple_of((g % 3) * CS, CS)

            def body(k, carry, base=base):
                m1, m2 = carry
                w1 = dup16(in_v[pl.ds(base + 32 * k, 16)])
                w2 = dup16(in_v[pl.ds(base + 32 * k + 16, 16)])
                for i in range(8):
                    m1, m2 = update(m1, m2, bc(w1[2 * i]), bc(w1[2 * i + 1]))
                for i in range(8):
                    m1, m2 = update(m1, m2, bc(w2[2 * i]), bc(w2[2 * i + 1]))
                return m1, m2

            init = (tuple(neg for _ in range(NV)),
                    tuple(neg for _ in range(NV)))
            m1, m2 = lax.fori_loop(0, (P - 16) // 16, body, init)
            w1 = dup16(in_v[pl.ds(base + CS - 32, 16)])
            w2 = dup16(in_v[pl.ds(base + CS - 16, 16)])
            for i in range(8):
                m1, m2 = update(m1, m2, bc(w1[2 * i]), bc(w1[2 * i + 1]))
            for i in range(7):
                m1, m2 = update(m1, m2, bc(w2[2 * i]), bc(w2[2 * i + 1]))
            dlast = jnp.where(dim1, w2[15], w2[14])
            m1, m2 = update(m1, m2, bc(w2[14]), bc(dlast))

            for j in range(NV):
                off = u * uout + 16 * j
                out_v[pl.ds(off, 16)] = plsc.bitcast(
                    jnp.maximum(m1[j], zero), jnp.uint32)
                out_v[pl.ds(off + TPAD // 2, 16)] = plsc.bitcast(
                    jnp.maximum(m2[j], zero), jnp.uint32)

        pltpu.sync_copy(out_v, out_hbm.at[pl.ds(wid * USC * uout, USC * uout)])

    return sc_kernel(pd_flat)


def _tc_body(pd_ref, out_ref):
    g = pl.program_id(0) + NU_SC
    dim1 = ((g // 3) % 2) == 1
    tvec = lax.broadcasted_iota(jnp.int32, (1, TPAD), 1).astype(
        jnp.float32) * (1.0 / (T - 1))

    def tent(w):
        return jnp.minimum(tvec - w[:, 0:1], w[:, 1:2] - tvec)

    def body(k, carry):
        m1, m2 = carry
        v = tent(pd_ref[0, 0, 0, pl.ds(8 * k, 8), :])
        nm2 = jnp.maximum(m2, jnp.minimum(m1, v))
        nm1 = jnp.maximum(m1, v)
        return nm1, nm2

    init = (jnp.full((8, TPAD), NEG, jnp.float32),
            jnp.full((8, TPAD), NEG, jnp.float32))
    m1, m2 = lax.fori_loop(0, P // 8 - 1, body, init)
    v = tent(pd_ref[0, 0, 0, pl.ds(P - 8, 8), :])
    keep = (lax.broadcasted_iota(jnp.int32, (8, TPAD), 0) < 7) | dim1
    v = jnp.where(keep, v, NEG)
    m2 = jnp.maximum(m2, jnp.minimum(m1, v))
    m1 = jnp.maximum(m1, v)

    while m1.shape[0] > 1:
        h = m1.shape[0] // 2
        c1, c2 = m1[h:], m2[h:]
        m1, m2 = m1[:h], m2[:h]
        nm2 = jnp.maximum(jnp.minimum(m1, c1), jnp.maximum(m2, c2))
        m1 = jnp.maximum(m1, c1)
        m2 = nm2
    out_ref[0, 0, :] = jnp.maximum(m1[0], 0.0)
    out_ref[0, 1, :] = jnp.maximum(m2[0], 0.0)


def _pl_tc_call(pd):
    def imap(g):
        gg = g + NU_SC
        return (gg // 6, (gg // 3) % 2, gg % 3, 0, 0)

    return pl.pallas_call(
        _tc_body,
        grid=(NU_TC,),
        in_specs=[pl.BlockSpec((1, 1, 1, P, 2), imap)],
        out_specs=pl.BlockSpec((1, KTOP, TPAD), lambda g: (g, 0, 0)),
        out_shape=jax.ShapeDtypeStruct((NU_TC, KTOP, TPAD), jnp.float32),
    )(pd)


@jax.jit
def kernel(pd):
    sc_u = _pl_sc_call(pd.reshape(B, D, C * P * 2))
    tc_f = _pl_tc_call(pd)
    sc_bf = lax.bitcast_convert_type(
        lax.bitcast_convert_type(
            sc_u.reshape(NU_SC, KTOP, TPAD // 2), jnp.uint16), jnp.bfloat16)
    sc_f = sc_bf.reshape(NU_SC, KTOP, TPAD).astype(jnp.float32)
    allu = jnp.concatenate([sc_f, tc_f], axis=0)
    out = allu.reshape(B, D, C, KTOP, TPAD)[..., :T]
    return out.transpose(0, 1, 3, 2, 4).reshape(B, D, KTOP, C * T)


# speedup vs baseline: 6.3546x; 6.3546x over previous
import functools

import jax
import jax.numpy as jnp
from jax import lax
from jax.experimental import pallas as pl
from jax.experimental.pallas import tpu as pltpu
from jax.experimental.pallas import tpu_sc as plsc

T = 100
TPAD = 128
NV = TPAD // 32
KTOP = 2
B, D, C, P = 32, 2, 3, 2048
NEG_DUP = 0xC000C000


def _rne_bf16_bits(f):
    u = plsc.bitcast(f, jnp.uint32)
    return (u + jnp.uint32(0x7FFF) + ((u >> jnp.uint32(16)) & jnp.uint32(1))) >> jnp.uint32(16)


def _pl_sc_call(pd_flat):
    mesh = plsc.VectorSubcoreMesh(core_axis_name="c", subcore_axis_name="s")

    slab = C * P * 2
    oslab = KTOP * C * (TPAD // 2)

    @functools.partial(
        pl.kernel,
        mesh=mesh,
        compiler_params=pltpu.CompilerParams(needs_layout_passes=False),
        out_type=jax.ShapeDtypeStruct((B * D * oslab,), jnp.uint32),
        scratch_types=[
            pltpu.VMEM((slab,), jnp.float32),
            pltpu.VMEM((oslab,), jnp.uint32),
        ],
    )
    def sc_kernel(pd_hbm, out_hbm, in_v, out_v):
        wid = lax.axis_index("s") * 2 + lax.axis_index("c")
        lane2 = lax.iota(jnp.int32, 16).astype(jnp.float32) * 2.0
        tvecs = []
        for j in range(NV):
            te = (lane2 + float(32 * j)) * (1.0 / (T - 1))
            to = (lane2 + float(32 * j + 1)) * (1.0 / (T - 1))
            tu = _rne_bf16_bits(te) | (_rne_bf16_bits(to) << jnp.uint32(16))
            tvecs.append(plsc.bitcast(tu, jnp.bfloat16))

        def dup16(w):
            r = _rne_bf16_bits(w)
            return r | (r << jnp.uint32(16))

        def bcast16(w, i):
            return plsc.bitcast(jnp.full((16,), w[i], jnp.uint32), jnp.bfloat16)

        def update(m1, m2, w, i):
            bv = bcast16(w, 2 * i)
            dv = bcast16(w, 2 * i + 1)
            nm1, nm2 = [], []
            for j in range(NV):
                v = jnp.minimum(tvecs[j] - bv, dv - tvecs[j])
                nm2.append(jnp.maximum(m2[j], jnp.minimum(m1[j], v)))
                nm1.append(jnp.maximum(m1[j], v))
            return tuple(nm1), tuple(nm2)

        neg = plsc.bitcast(jnp.full((16,), NEG_DUP, jnp.uint32), jnp.bfloat16)
        zero = plsc.bitcast(jnp.full((16,), 0, jnp.uint32), jnp.bfloat16)

        def run8(m1, m2, off, n):
            w = dup16(in_v[pl.ds(off, 16)])
            for i in range(n):
                m1, m2 = update(m1, m2, w, i)
            return m1, m2

        for d in range(D):
            pltpu.sync_copy(pd_hbm.at[wid, d], in_v)
            nbars = P - 1 if d == 0 else P
            nfull = nbars // 32
            ntail = nbars % 32

            for c in range(C):
                base = c * P * 2

                def body(k, carry, base=base):
                    m1, m2 = carry
                    for q in range(4):
                        m1, m2 = run8(m1, m2, base + 64 * k + 16 * q, 8)
                    return m1, m2

                init = (tuple(neg for _ in range(NV)),
                        tuple(neg for _ in range(NV)))
                m1, m2 = lax.fori_loop(0, nfull, body, init)
                left = ntail
                off = base + 64 * nfull
                while left > 0:
                    m1, m2 = run8(m1, m2, off, min(left, 8))
                    off += 16
                    left -= 8

                for j in range(NV):
                    off = c * (TPAD // 2) + 16 * j
                    out_v[pl.ds(off, 16)] = plsc.bitcast(
                        jnp.maximum(m1[j], zero), jnp.uint32)
                    out_v[pl.ds(C * (TPAD // 2) + off, 16)] = plsc.bitcast(
                        jnp.maximum(m2[j], zero), jnp.uint32)

            pltpu.sync_copy(
                out_v, out_hbm.at[pl.ds((wid * D + d) * oslab, oslab)])

    return sc_kernel(pd_flat)


@jax.jit
def kernel(pd):
    out_u = _pl_sc_call(pd.reshape(B, D, C * P * 2)).reshape(
        B, D, KTOP, C, TPAD // 2)
    out_bf = lax.bitcast_convert_type(
        lax.bitcast_convert_type(out_u, jnp.uint16), jnp.bfloat16)
    out = out_bf.reshape(B, D, KTOP, C, TPAD).astype(jnp.float32)
    return out[..., :T].reshape(B, D, KTOP, C * T)
